# Initial kernel scaffold; baseline (speedup 1.0000x reference)
import jax
import jax.numpy as jnp
from jax.experimental import pallas as pl


def kernel(x, emb_weight):
    raise NotImplementedError("write your pallas kernel here")


# speedup vs baseline: 1.6660x; 1.6660x over previous
import jax
import jax.numpy as jnp
from jax.experimental import pallas as pl

_BS = 1024


def _add_body(x_ref, emb_ref, out_ref):
    out_ref[...] = x_ref[...] + emb_ref[...]


def kernel(x, emb_weight):
    B, S, D = x.shape
    bs = min(_BS, S)
    grid = (S // bs, B)
    return pl.pallas_call(
        _add_body,
        grid=grid,
        in_specs=[
            pl.BlockSpec((1, bs, D), lambda s, b: (b, s, 0)),
            pl.BlockSpec((bs, D), lambda s, b: (s, 0)),
        ],
        out_specs=pl.BlockSpec((1, bs, D), lambda s, b: (b, s, 0)),
        out_shape=jax.ShapeDtypeStruct((B, S, D), x.dtype),
    )(x, emb_weight)


# speedup vs baseline: 1.7365x; 1.0423x over previous
import jax
import jax.numpy as jnp
from jax.experimental import pallas as pl

_BS = 2048


def _add_body(x_ref, emb_ref, out_ref):
    out_ref[...] = x_ref[...] + emb_ref[...]


def kernel(x, emb_weight):
    B, S, D = x.shape
    bs = min(_BS, S)
    grid = (S // bs, B)
    return pl.pallas_call(
        _add_body,
        grid=grid,
        in_specs=[
            pl.BlockSpec((1, bs, D), lambda s, b: (b, s, 0)),
            pl.BlockSpec((bs, D), lambda s, b: (s, 0)),
        ],
        out_specs=pl.BlockSpec((1, bs, D), lambda s, b: (b, s, 0)),
        out_shape=jax.ShapeDtypeStruct((B, S, D), x.dtype),
    )(x, emb_weight)
